# baseline (speedup 1.0000x reference)
import functools

import jax
import jax.numpy as jnp
import numpy as np
from jax import lax
from jax.experimental import pallas as pl
from jax.experimental.pallas import tpu as pltpu
from jax.experimental.pallas import tpu_sc as plsc

_NW = 32
_NC = 2
_LANES = 16
_MAGIC = np.float32(1.5 * 2.0**23)
_RSQRT_MAGIC = np.int32(0x5F3759DF)

_C3 = np.float32(1.0 / 3.0)
_C5 = np.float32(1.0 / 5.0)
_C6 = np.float32(1.0 / 6.0)


def _sc_energy(coords, pairs, const16, c6, b, n_pairs, chunk):
    per_w = n_pairs // _NW
    n_chunks = per_w // chunk
    mesh = plsc.VectorSubcoreMesh(core_axis_name="c", subcore_axis_name="s")

    @functools.partial(
        pl.kernel,
        out_type=jax.ShapeDtypeStruct((n_pairs,), jnp.float32),
        mesh=mesh,
        compiler_params=pltpu.CompilerParams(
            needs_layout_passes=False, use_tc_tiling_on_sc=False),
        scratch_types=[
            pltpu.VMEM((chunk, 2), jnp.int32),
            pltpu.VMEM((2 * chunk,), jnp.int32),
            pltpu.VMEM((2 * chunk, 3), jnp.float32),
            pltpu.VMEM((chunk,), jnp.float32),
            pltpu.VMEM((chunk,), jnp.float32),
            pltpu.VMEM((chunk,), jnp.float32),
            pltpu.VMEM((8, 16), jnp.float32),
            pltpu.SemaphoreType.DMA,
        ],
    )
    def run(coords_hbm, pairs_hbm, const_hbm, c6_hbm, b_hbm, ene_hbm,
            pr_v, idx_v, rows_v, c6_v, b_v, out_v, const_v, sem):
        wid = lax.axis_index("s") * _NC + lax.axis_index("c")
        pltpu.sync_copy(const_hbm, const_v)
        bx = const_v[0]
        by = const_v[1]
        bz = const_v[2]
        ibx = const_v[3]
        iby = const_v[4]
        ibz = const_v[5]
        cut = const_v[6]
        base = wid * per_w
        lanes = lax.iota(jnp.int32, _LANES)
        c0 = lanes * 0
        c1 = c0 + 1
        c2c = c0 + 2
        half = np.float32(0.5)
        threehalf = np.float32(1.5)

        def do_chunk(ci, carry):
            pbase = base + ci * chunk
            pltpu.sync_copy(pairs_hbm.at[pl.ds(pbase, chunk), :], pr_v)
            pltpu.sync_copy(c6_hbm.at[pl.ds(pbase, chunk)], c6_v)
            pltpu.sync_copy(b_hbm.at[pl.ds(pbase, chunk)], b_v)

            def build(j, c2):
                p = j * _LANES + lanes
                i0 = plsc.load_gather(pr_v, [p, c0])
                i1 = plsc.load_gather(pr_v, [p, c1])
                idx_v[pl.ds(j * _LANES, _LANES)] = i0
                idx_v[pl.ds(chunk + j * _LANES, _LANES)] = i1
                return c2

            lax.fori_loop(0, chunk // _LANES, build, 0)
            pltpu.async_copy(coords_hbm.at[idx_v], rows_v, sem).wait()

            def inner(j, c2):
                p = j * _LANES + lanes
                r0 = p
                r1 = chunk + p
                x0 = plsc.load_gather(rows_v, [r0, c0])
                x1 = plsc.load_gather(rows_v, [r1, c0])
                y0 = plsc.load_gather(rows_v, [r0, c1])
                y1 = plsc.load_gather(rows_v, [r1, c1])
                z0 = plsc.load_gather(rows_v, [r0, c2c])
                z1 = plsc.load_gather(rows_v, [r1, c2c])
                dx = x1 - x0
                dy = y1 - y0
                dz = z1 - z0
                kx = (dx * ibx + _MAGIC) - _MAGIC
                ky = (dy * iby + _MAGIC) - _MAGIC
                kz = (dz * ibz + _MAGIC) - _MAGIC
                dx = dx - bx * kx
                dy = dy - by * ky
                dz = dz - bz * kz
                r2 = (dx * dx + dy * dy) + dz * dz
                yv = plsc.bitcast(
                    _RSQRT_MAGIC - lax.shift_right_arithmetic(
                        plsc.bitcast(r2, jnp.int32), 1), jnp.float32)
                h = half * r2
                yv = yv * (threehalf - h * yv * yv)
                yv = yv * (threehalf - h * yv * yv)
                yv = yv * (threehalf - h * yv * yv)
                r = r2 * yv
                bv = b_v[pl.ds(j * _LANES, _LANES)]
                cv = c6_v[pl.ds(j * _LANES, _LANES)]
                u = bv * r
                t6 = 1.0 + u * _C6
                t5 = 1.0 + (u * _C5) * t6
                t4 = 1.0 + (u * np.float32(0.25)) * t5
                t3 = 1.0 + (u * _C3) * t4
                t2 = 1.0 + (u * half) * t3
                s = 1.0 + u * t2
                f6 = 1.0 - jnp.exp(-u) * s
                num = -(cv * f6)
                a = r * r
                a2 = a * a
                r6 = a * a2
                ene = num / r6
                out_v[pl.ds(j * _LANES, _LANES)] = jnp.where(
                    r <= cut, ene, np.float32(0.0))
                return c2

            lax.fori_loop(0, chunk // _LANES, inner, 0)
            pltpu.sync_copy(out_v, ene_hbm.at[pl.ds(pbase, chunk)])
            return carry

        lax.fori_loop(0, n_chunks, do_chunk, 0)

    return run(coords, pairs, const16, c6, b)


def kernel(coords, pairs, box, c6, b, cutoff):
    n_pairs = pairs.shape[0]
    boxf = box.astype(jnp.float32)
    cutf = jnp.asarray(cutoff, jnp.float32).reshape(1)
    const8 = jnp.concatenate(
        [boxf, 1.0 / boxf, cutf, jnp.zeros((1,), jnp.float32)])
    const16 = jnp.broadcast_to(const8[:, None], (8, 16))
    return _sc_energy(coords, pairs, const16, c6, b, n_pairs, chunk=4000)


# speedup vs baseline: 1.1784x; 1.1784x over previous
import functools

import jax
import jax.numpy as jnp
import numpy as np
from jax import lax
from jax.experimental import pallas as pl
from jax.experimental.pallas import tpu as pltpu
from jax.experimental.pallas import tpu_sc as plsc

_NW = 32
_NC = 2
_LANES = 16
_MAGIC = np.float32(1.5 * 2.0**23)
_RSQRT_MAGIC = np.int32(0x5F3759DF)

_C3 = np.float32(1.0 / 3.0)
_C5 = np.float32(1.0 / 5.0)
_C6 = np.float32(1.0 / 6.0)


def _sc_energy(coords, pairs, const16, c6, b, n_pairs, chunk):
    per_w = n_pairs // _NW
    n_chunks = per_w // chunk
    mesh = plsc.VectorSubcoreMesh(core_axis_name="c", subcore_axis_name="s")

    @functools.partial(
        pl.kernel,
        out_type=jax.ShapeDtypeStruct((n_pairs,), jnp.float32),
        mesh=mesh,
        compiler_params=pltpu.CompilerParams(
            needs_layout_passes=False, use_tc_tiling_on_sc=False),
        scratch_types=[
            pltpu.VMEM((2 * chunk,), jnp.int32),
            pltpu.VMEM((2 * chunk, 3), jnp.float32),
            pltpu.VMEM((chunk,), jnp.float32),
            pltpu.VMEM((chunk,), jnp.float32),
            pltpu.VMEM((chunk,), jnp.float32),
            pltpu.VMEM((8, 16), jnp.float32),
            pltpu.SemaphoreType.DMA,
        ],
    )
    def run(coords_hbm, pairs_hbm, const_hbm, c6_hbm, b_hbm, ene_hbm,
            idx_v, rows_v, c6_v, b_v, out_v, const_v, sem):
        wid = lax.axis_index("s") * _NC + lax.axis_index("c")
        pltpu.sync_copy(const_hbm, const_v)
        bx = const_v[0]
        by = const_v[1]
        bz = const_v[2]
        ibx = const_v[3]
        iby = const_v[4]
        ibz = const_v[5]
        cut = const_v[6]
        base = wid * per_w
        lanes = lax.iota(jnp.int32, _LANES)
        c0 = lanes * 0
        c1 = c0 + 1
        c2c = c0 + 2
        half = np.float32(0.5)
        threehalf = np.float32(1.5)

        def do_chunk(ci, carry):
            pbase = base + ci * chunk
            pltpu.sync_copy(pairs_hbm.at[pl.ds(2 * pbase, 2 * chunk)], idx_v)
            pltpu.sync_copy(c6_hbm.at[pl.ds(pbase, chunk)], c6_v)
            pltpu.sync_copy(b_hbm.at[pl.ds(pbase, chunk)], b_v)
            pltpu.async_copy(coords_hbm.at[idx_v], rows_v, sem).wait()

            def inner(j, c2):
                p = j * _LANES + lanes
                r0 = p + p
                r1 = r0 + 1
                x0 = plsc.load_gather(rows_v, [r0, c0])
                x1 = plsc.load_gather(rows_v, [r1, c0])
                y0 = plsc.load_gather(rows_v, [r0, c1])
                y1 = plsc.load_gather(rows_v, [r1, c1])
                z0 = plsc.load_gather(rows_v, [r0, c2c])
                z1 = plsc.load_gather(rows_v, [r1, c2c])
                dx = x1 - x0
                dy = y1 - y0
                dz = z1 - z0
                kx = (dx * ibx + _MAGIC) - _MAGIC
                ky = (dy * iby + _MAGIC) - _MAGIC
                kz = (dz * ibz + _MAGIC) - _MAGIC
                dx = dx - bx * kx
                dy = dy - by * ky
                dz = dz - bz * kz
                r2 = (dx * dx + dy * dy) + dz * dz
                yv = plsc.bitcast(
                    _RSQRT_MAGIC - lax.shift_right_arithmetic(
                        plsc.bitcast(r2, jnp.int32), 1), jnp.float32)
                h = half * r2
                yv = yv * (threehalf - h * yv * yv)
                yv = yv * (threehalf - h * yv * yv)
                yv = yv * (threehalf - h * yv * yv)
                r = r2 * yv
                bv = b_v[pl.ds(j * _LANES, _LANES)]
                cv = c6_v[pl.ds(j * _LANES, _LANES)]
                u = bv * r
                t6 = 1.0 + u * _C6
                t5 = 1.0 + (u * _C5) * t6
                t4 = 1.0 + (u * np.float32(0.25)) * t5
                t3 = 1.0 + (u * _C3) * t4
                t2 = 1.0 + (u * half) * t3
                s = 1.0 + u * t2
                f6 = 1.0 - jnp.exp(-u) * s
                num = -(cv * f6)
                a = r * r
                a2 = a * a
                r6 = a * a2
                ene = num / r6
                out_v[pl.ds(j * _LANES, _LANES)] = jnp.where(
                    r <= cut, ene, np.float32(0.0))
                return c2

            lax.fori_loop(0, chunk // _LANES, inner, 0)
            pltpu.sync_copy(out_v, ene_hbm.at[pl.ds(pbase, chunk)])
            return carry

        lax.fori_loop(0, n_chunks, do_chunk, 0)

    return run(coords, pairs.reshape(-1), const16, c6, b)


def kernel(coords, pairs, box, c6, b, cutoff):
    n_pairs = pairs.shape[0]
    boxf = box.astype(jnp.float32)
    cutf = jnp.asarray(cutoff, jnp.float32).reshape(1)
    const8 = jnp.concatenate(
        [boxf, 1.0 / boxf, cutf, jnp.zeros((1,), jnp.float32)])
    const16 = jnp.broadcast_to(const8[:, None], (8, 16))
    return _sc_energy(coords, pairs, const16, c6, b, n_pairs, chunk=4000)


# speedup vs baseline: 1.2288x; 1.0428x over previous
import functools

import jax
import jax.numpy as jnp
import numpy as np
from jax import lax
from jax.experimental import pallas as pl
from jax.experimental.pallas import tpu as pltpu
from jax.experimental.pallas import tpu_sc as plsc

_NW = 32
_NC = 2
_LANES = 16
_MAGIC = np.float32(1.5 * 2.0**23)
_RSQRT_MAGIC = np.int32(0x5F3759DF)

_C3 = np.float32(1.0 / 3.0)
_C5 = np.float32(1.0 / 5.0)
_C6 = np.float32(1.0 / 6.0)


def _sc_energy(coords, pairs, const16, c6, b, n_pairs, chunk):
    per_w = n_pairs // _NW
    n_chunks = per_w // chunk
    mesh = plsc.VectorSubcoreMesh(core_axis_name="c", subcore_axis_name="s")

    @functools.partial(
        pl.kernel,
        out_type=jax.ShapeDtypeStruct((n_pairs,), jnp.float32),
        mesh=mesh,
        compiler_params=pltpu.CompilerParams(
            needs_layout_passes=False, use_tc_tiling_on_sc=False),
        scratch_types=[
            pltpu.VMEM((2 * chunk,), jnp.int32),
            pltpu.VMEM((2 * chunk,), jnp.int32),
            pltpu.VMEM((2 * chunk, 3), jnp.float32),
            pltpu.VMEM((2 * chunk, 3), jnp.float32),
            pltpu.VMEM((chunk,), jnp.float32),
            pltpu.VMEM((chunk,), jnp.float32),
            pltpu.VMEM((chunk,), jnp.float32),
            pltpu.VMEM((chunk,), jnp.float32),
            pltpu.VMEM((chunk,), jnp.float32),
            pltpu.VMEM((chunk,), jnp.float32),
            pltpu.VMEM((8, 16), jnp.float32),
            pltpu.SemaphoreType.DMA,
            pltpu.SemaphoreType.DMA,
        ],
    )
    def run(coords_hbm, pairs_hbm, const_hbm, c6_hbm, b_hbm, ene_hbm,
            idx_a, idx_b, rows_a, rows_b, c6_a, c6_b, b_a, b_b,
            out_a, out_b, const_v, sem_a, sem_b):
        wid = lax.axis_index("s") * _NC + lax.axis_index("c")
        pltpu.sync_copy(const_hbm, const_v)
        bx = const_v[0]
        by = const_v[1]
        bz = const_v[2]
        ibx = const_v[3]
        iby = const_v[4]
        ibz = const_v[5]
        cut = const_v[6]
        base = wid * per_w
        lanes = lax.iota(jnp.int32, _LANES)
        c0 = lanes * 0
        c1 = c0 + 1
        c2c = c0 + 2
        half = np.float32(0.5)
        threehalf = np.float32(1.5)

        def stage(pbase, idx_v, rows_v, c6_v, b_v, sem):
            pltpu.sync_copy(pairs_hbm.at[pl.ds(2 * pbase, 2 * chunk)], idx_v)
            pltpu.sync_copy(c6_hbm.at[pl.ds(pbase, chunk)], c6_v)
            pltpu.sync_copy(b_hbm.at[pl.ds(pbase, chunk)], b_v)
            return pltpu.async_copy(coords_hbm.at[idx_v], rows_v, sem)

        def compute(pbase, rows_v, c6_v, b_v, out_v):
            def inner(j, c2):
                p = j * _LANES + lanes
                r0 = p + p
                r1 = r0 + 1
                x0 = plsc.load_gather(rows_v, [r0, c0])
                x1 = plsc.load_gather(rows_v, [r1, c0])
                y0 = plsc.load_gather(rows_v, [r0, c1])
                y1 = plsc.load_gather(rows_v, [r1, c1])
                z0 = plsc.load_gather(rows_v, [r0, c2c])
                z1 = plsc.load_gather(rows_v, [r1, c2c])
                dx = x1 - x0
                dy = y1 - y0
                dz = z1 - z0
                kx = (dx * ibx + _MAGIC) - _MAGIC
                ky = (dy * iby + _MAGIC) - _MAGIC
                kz = (dz * ibz + _MAGIC) - _MAGIC
                dx = dx - bx * kx
                dy = dy - by * ky
                dz = dz - bz * kz
                r2 = (dx * dx + dy * dy) + dz * dz
                yv = plsc.bitcast(
                    _RSQRT_MAGIC - lax.shift_right_arithmetic(
                        plsc.bitcast(r2, jnp.int32), 1), jnp.float32)
                h = half * r2
                yv = yv * (threehalf - h * yv * yv)
                yv = yv * (threehalf - h * yv * yv)
                yv = yv * (threehalf - h * yv * yv)
                r = r2 * yv
                bv = b_v[pl.ds(j * _LANES, _LANES)]
                cv = c6_v[pl.ds(j * _LANES, _LANES)]
                u = bv * r
                t6 = 1.0 + u * _C6
                t5 = 1.0 + (u * _C5) * t6
                t4 = 1.0 + (u * np.float32(0.25)) * t5
                t3 = 1.0 + (u * _C3) * t4
                t2 = 1.0 + (u * half) * t3
                s = 1.0 + u * t2
                f6 = 1.0 - jnp.exp(-u) * s
                num = -(cv * f6)
                a = r * r
                a2 = a * a
                r6 = a * a2
                ene = num / r6
                out_v[pl.ds(j * _LANES, _LANES)] = jnp.where(
                    r <= cut, ene, np.float32(0.0))
                return c2

            lax.fori_loop(0, chunk // _LANES, inner, 0)
            pltpu.sync_copy(out_v, ene_hbm.at[pl.ds(pbase, chunk)])

        def do_pair(k, carry):
            pb0 = base + (2 * k) * chunk
            pb1 = pb0 + chunk
            ga = stage(pb0, idx_a, rows_a, c6_a, b_a, sem_a)
            gb = stage(pb1, idx_b, rows_b, c6_b, b_b, sem_b)
            ga.wait()
            compute(pb0, rows_a, c6_a, b_a, out_a)
            gb.wait()
            compute(pb1, rows_b, c6_b, b_b, out_b)
            return carry

        lax.fori_loop(0, n_chunks // 2, do_pair, 0)

    return run(coords, pairs.reshape(-1), const16, c6, b)


def kernel(coords, pairs, box, c6, b, cutoff):
    n_pairs = pairs.shape[0]
    boxf = box.astype(jnp.float32)
    cutf = jnp.asarray(cutoff, jnp.float32).reshape(1)
    const8 = jnp.concatenate(
        [boxf, 1.0 / boxf, cutf, jnp.zeros((1,), jnp.float32)])
    const16 = jnp.broadcast_to(const8[:, None], (8, 16))
    return _sc_energy(coords, pairs, const16, c6, b, n_pairs, chunk=2000)


# speedup vs baseline: 6.7551x; 5.4971x over previous
import functools

import jax
import jax.numpy as jnp
import numpy as np
from jax import lax
from jax.experimental import pallas as pl
from jax.experimental.pallas import tpu as pltpu
from jax.experimental.pallas import tpu_sc as plsc

_NW = 32
_NC = 2
_LANES = 16
_MAGIC = np.float32(1.5 * 2.0**23)
_RSQRT_MAGIC = np.int32(0x5F3759DF)

_C3 = np.float32(1.0 / 3.0)
_C5 = np.float32(1.0 / 5.0)
_C6 = np.float32(1.0 / 6.0)


def _sc_energy(xs, ys, zs, i0, i1, const16, c6, b, n_pairs, chunk):
    per_w = n_pairs // _NW
    n_chunks = per_w // chunk
    mesh = plsc.VectorSubcoreMesh(core_axis_name="c", subcore_axis_name="s")

    def buf_set():
        return [
            pltpu.VMEM((chunk,), jnp.int32),
            pltpu.VMEM((chunk,), jnp.int32),
            pltpu.VMEM((chunk,), jnp.float32),
            pltpu.VMEM((chunk,), jnp.float32),
            pltpu.VMEM((chunk,), jnp.float32),
            pltpu.VMEM((chunk,), jnp.float32),
            pltpu.VMEM((chunk,), jnp.float32),
            pltpu.VMEM((chunk,), jnp.float32),
            pltpu.VMEM((chunk,), jnp.float32),
            pltpu.VMEM((chunk,), jnp.float32),
            pltpu.VMEM((chunk,), jnp.float32),
        ] + [pltpu.SemaphoreType.DMA] * 6

    @functools.partial(
        pl.kernel,
        out_type=jax.ShapeDtypeStruct((n_pairs,), jnp.float32),
        mesh=mesh,
        compiler_params=pltpu.CompilerParams(
            needs_layout_passes=False, use_tc_tiling_on_sc=False),
        scratch_types=buf_set() + buf_set() + [
            pltpu.VMEM((8, 16), jnp.float32),
        ],
    )
    def run(x_hbm, y_hbm, z_hbm, i0_hbm, i1_hbm, const_hbm, c6_hbm, b_hbm,
            ene_hbm, *scr):
        bufs_a = scr[:17]
        bufs_b = scr[17:34]
        const_v = scr[34]
        wid = lax.axis_index("s") * _NC + lax.axis_index("c")
        pltpu.sync_copy(const_hbm, const_v)
        bx = const_v[0]
        by = const_v[1]
        bz = const_v[2]
        ibx = const_v[3]
        iby = const_v[4]
        ibz = const_v[5]
        cut = const_v[6]
        base = wid * per_w
        half = np.float32(0.5)
        threehalf = np.float32(1.5)

        def stage(pbase, bufs):
            (i0_v, i1_v, x0_v, x1_v, y0_v, y1_v, z0_v, z1_v,
             c6_v, b_v, _out, s0, s1, s2, s3, s4, s5) = bufs
            sl = pl.ds(pbase, chunk)
            pltpu.sync_copy(i0_hbm.at[sl], i0_v)
            pltpu.sync_copy(i1_hbm.at[sl], i1_v)
            pltpu.sync_copy(c6_hbm.at[sl], c6_v)
            pltpu.sync_copy(b_hbm.at[sl], b_v)
            return [
                pltpu.async_copy(x_hbm.at[i0_v], x0_v, s0),
                pltpu.async_copy(x_hbm.at[i1_v], x1_v, s1),
                pltpu.async_copy(y_hbm.at[i0_v], y0_v, s2),
                pltpu.async_copy(y_hbm.at[i1_v], y1_v, s3),
                pltpu.async_copy(z_hbm.at[i0_v], z0_v, s4),
                pltpu.async_copy(z_hbm.at[i1_v], z1_v, s5),
            ]

        def compute(pbase, bufs):
            (_i0, _i1, x0_v, x1_v, y0_v, y1_v, z0_v, z1_v,
             c6_v, b_v, out_v, *_sems) = bufs

            def inner(j, c2):
                s = pl.ds(j * _LANES, _LANES)
                dx = x1_v[s] - x0_v[s]
                dy = y1_v[s] - y0_v[s]
                dz = z1_v[s] - z0_v[s]
                kx = (dx * ibx + _MAGIC) - _MAGIC
                ky = (dy * iby + _MAGIC) - _MAGIC
                kz = (dz * ibz + _MAGIC) - _MAGIC
                dx = dx - bx * kx
                dy = dy - by * ky
                dz = dz - bz * kz
                r2 = (dx * dx + dy * dy) + dz * dz
                yv = plsc.bitcast(
                    _RSQRT_MAGIC - lax.shift_right_arithmetic(
                        plsc.bitcast(r2, jnp.int32), 1), jnp.float32)
                h = half * r2
                yv = yv * (threehalf - h * yv * yv)
                yv = yv * (threehalf - h * yv * yv)
                yv = yv * (threehalf - h * yv * yv)
                r = r2 * yv
                bv = b_v[s]
                cv = c6_v[s]
                u = bv * r
                t6 = 1.0 + u * _C6
                t5 = 1.0 + (u * _C5) * t6
                t4 = 1.0 + (u * np.float32(0.25)) * t5
                t3 = 1.0 + (u * _C3) * t4
                t2 = 1.0 + (u * half) * t3
                sp = 1.0 + u * t2
                f6 = 1.0 - jnp.exp(-u) * sp
                num = -(cv * f6)
                a = r * r
                a2 = a * a
                r6 = a * a2
                ene = num / r6
                out_v[s] = jnp.where(r <= cut, ene, np.float32(0.0))
                return c2

            lax.fori_loop(0, chunk // _LANES, inner, 0)
            pltpu.sync_copy(out_v, ene_hbm.at[pl.ds(pbase, chunk)])

        def do_pair(k, carry):
            pb0 = base + (2 * k) * chunk
            pb1 = pb0 + chunk
            ga = stage(pb0, bufs_a)
            gb = stage(pb1, bufs_b)
            for g in ga:
                g.wait()
            compute(pb0, bufs_a)
            for g in gb:
                g.wait()
            compute(pb1, bufs_b)
            return carry

        lax.fori_loop(0, n_chunks // 2, do_pair, 0)

    return run(xs, ys, zs, i0, i1, const16, c6, b)


def kernel(coords, pairs, box, c6, b, cutoff):
    n_pairs = pairs.shape[0]
    boxf = box.astype(jnp.float32)
    cutf = jnp.asarray(cutoff, jnp.float32).reshape(1)
    const8 = jnp.concatenate(
        [boxf, 1.0 / boxf, cutf, jnp.zeros((1,), jnp.float32)])
    const16 = jnp.broadcast_to(const8[:, None], (8, 16))
    xs = coords[:, 0]
    ys = coords[:, 1]
    zs = coords[:, 2]
    i0 = pairs[:, 0]
    i1 = pairs[:, 1]
    return _sc_energy(xs, ys, zs, i0, i1, const16, c6, b, n_pairs, chunk=4000)


# speedup vs baseline: 6.8263x; 1.0105x over previous
import functools

import jax
import jax.numpy as jnp
import numpy as np
from jax import lax
from jax.experimental import pallas as pl
from jax.experimental.pallas import tpu as pltpu
from jax.experimental.pallas import tpu_sc as plsc

_NW = 32
_NC = 2
_LANES = 16
_MAGIC = np.float32(1.5 * 2.0**23)
_RSQRT_MAGIC = np.int32(0x5F3759DF)

_C3 = np.float32(1.0 / 3.0)
_C5 = np.float32(1.0 / 5.0)
_C6 = np.float32(1.0 / 6.0)


_W = 5


def _sc_energy(xs, ys, zs, i0, i1, const16, c6, b, n_pairs, chunk):
    per_w = n_pairs // _NW
    n_chunks = per_w // chunk
    mesh = plsc.VectorSubcoreMesh(core_axis_name="c", subcore_axis_name="s")

    def buf_set():
        return [
            pltpu.VMEM((chunk,), jnp.int32),
            pltpu.VMEM((chunk,), jnp.int32),
            pltpu.VMEM((chunk,), jnp.float32),
            pltpu.VMEM((chunk,), jnp.float32),
            pltpu.VMEM((chunk,), jnp.float32),
            pltpu.VMEM((chunk,), jnp.float32),
            pltpu.VMEM((chunk,), jnp.float32),
            pltpu.VMEM((chunk,), jnp.float32),
            pltpu.VMEM((chunk,), jnp.float32),
            pltpu.VMEM((chunk,), jnp.float32),
            pltpu.VMEM((chunk,), jnp.float32),
        ] + [pltpu.SemaphoreType.DMA] * 6

    @functools.partial(
        pl.kernel,
        out_type=jax.ShapeDtypeStruct((n_pairs,), jnp.float32),
        mesh=mesh,
        compiler_params=pltpu.CompilerParams(
            needs_layout_passes=False, use_tc_tiling_on_sc=False),
        scratch_types=sum([buf_set() for _ in range(_W)], []) + [
            pltpu.VMEM((8, 16), jnp.float32),
        ],
    )
    def run(x_hbm, y_hbm, z_hbm, i0_hbm, i1_hbm, const_hbm, c6_hbm, b_hbm,
            ene_hbm, *scr):
        bufs = [scr[17 * i:17 * (i + 1)] for i in range(_W)]
        const_v = scr[17 * _W]
        wid = lax.axis_index("s") * _NC + lax.axis_index("c")
        pltpu.sync_copy(const_hbm, const_v)
        bx = const_v[0]
        by = const_v[1]
        bz = const_v[2]
        ibx = const_v[3]
        iby = const_v[4]
        ibz = const_v[5]
        cut = const_v[6]
        base = wid * per_w
        half = np.float32(0.5)
        threehalf = np.float32(1.5)

        def stage(pbase, bufs):
            (i0_v, i1_v, x0_v, x1_v, y0_v, y1_v, z0_v, z1_v,
             c6_v, b_v, _out, s0, s1, s2, s3, s4, s5) = bufs
            sl = pl.ds(pbase, chunk)
            pltpu.sync_copy(i0_hbm.at[sl], i0_v)
            pltpu.sync_copy(i1_hbm.at[sl], i1_v)
            pltpu.sync_copy(c6_hbm.at[sl], c6_v)
            pltpu.sync_copy(b_hbm.at[sl], b_v)
            return [
                pltpu.async_copy(x_hbm.at[i0_v], x0_v, s0),
                pltpu.async_copy(x_hbm.at[i1_v], x1_v, s1),
                pltpu.async_copy(y_hbm.at[i0_v], y0_v, s2),
                pltpu.async_copy(y_hbm.at[i1_v], y1_v, s3),
                pltpu.async_copy(z_hbm.at[i0_v], z0_v, s4),
                pltpu.async_copy(z_hbm.at[i1_v], z1_v, s5),
            ]

        def compute(pbase, bufs):
            (_i0, _i1, x0_v, x1_v, y0_v, y1_v, z0_v, z1_v,
             c6_v, b_v, out_v, *_sems) = bufs

            def inner(j, c2):
                s = pl.ds(j * _LANES, _LANES)
                dx = x1_v[s] - x0_v[s]
                dy = y1_v[s] - y0_v[s]
                dz = z1_v[s] - z0_v[s]
                kx = (dx * ibx + _MAGIC) - _MAGIC
                ky = (dy * iby + _MAGIC) - _MAGIC
                kz = (dz * ibz + _MAGIC) - _MAGIC
                dx = dx - bx * kx
                dy = dy - by * ky
                dz = dz - bz * kz
                r2 = (dx * dx + dy * dy) + dz * dz
                yv = plsc.bitcast(
                    _RSQRT_MAGIC - lax.shift_right_arithmetic(
                        plsc.bitcast(r2, jnp.int32), 1), jnp.float32)
                h = half * r2
                yv = yv * (threehalf - h * yv * yv)
                yv = yv * (threehalf - h * yv * yv)
                yv = yv * (threehalf - h * yv * yv)
                r = r2 * yv
                bv = b_v[s]
                cv = c6_v[s]
                u = bv * r
                t6 = 1.0 + u * _C6
                t5 = 1.0 + (u * _C5) * t6
                t4 = 1.0 + (u * np.float32(0.25)) * t5
                t3 = 1.0 + (u * _C3) * t4
                t2 = 1.0 + (u * half) * t3
                sp = 1.0 + u * t2
                f6 = 1.0 - jnp.exp(-u) * sp
                num = -(cv * f6)
                a = r * r
                a2 = a * a
                r6 = a * a2
                ene = num / r6
                out_v[s] = jnp.where(r <= cut, ene, np.float32(0.0))
                return c2

            lax.fori_loop(0, chunk // _LANES, inner, 0)
            pltpu.sync_copy(out_v, ene_hbm.at[pl.ds(pbase, chunk)])

        def do_win(k, carry):
            pb = base + k * (_W * chunk)
            descs = [None] * _W
            descs[0] = stage(pb, bufs[0])
            descs[1] = stage(pb + chunk, bufs[1])
            for i in range(_W):
                if i + 2 < _W:
                    descs[i + 2] = stage(pb + (i + 2) * chunk, bufs[i + 2])
                for g in descs[i]:
                    g.wait()
                compute(pb + i * chunk, bufs[i])
            return carry

        lax.fori_loop(0, n_chunks // _W, do_win, 0)

    return run(xs, ys, zs, i0, i1, const16, c6, b)


def kernel(coords, pairs, box, c6, b, cutoff):
    n_pairs = pairs.shape[0]
    boxf = box.astype(jnp.float32)
    cutf = jnp.asarray(cutoff, jnp.float32).reshape(1)
    const8 = jnp.concatenate(
        [boxf, 1.0 / boxf, cutf, jnp.zeros((1,), jnp.float32)])
    const16 = jnp.broadcast_to(const8[:, None], (8, 16))
    xs = coords[:, 0]
    ys = coords[:, 1]
    zs = coords[:, 2]
    i0 = pairs[:, 0]
    i1 = pairs[:, 1]
    return _sc_energy(xs, ys, zs, i0, i1, const16, c6, b, n_pairs, chunk=2000)
